# baseline (speedup 1.0000x reference)
import jax
import jax.numpy as jnp
from jax import lax
from jax.experimental import pallas as pl
from jax.experimental.pallas import tpu as pltpu

IMW = 224
ROWS = 8 * 192 * 224
BR = 8192
NBLK = ROWS // BR
K = 3


def _body(x_hbm, p_ref, o_hbm,
          in0, in1, in2, out0, out1, out2,
          is0, is1, is2, os0, os1, os2):
    ins = (in0, in1, in2)
    outs = (out0, out1, out2)
    isems = (is0, is1, is2)
    osems = (os0, os1, os2)
    perm = p_ref[...]

    def start_in(i, b):
        pltpu.make_async_copy(
            x_hbm.at[pl.ds(i * BR, BR)], ins[b], isems[b]).start()

    for b in range(K):
        start_in(b, b)

    def group_body(g, _):
        for b in range(K):
            i = g * K + b
            pltpu.make_async_copy(
                x_hbm.at[pl.ds(0, BR)], ins[b], isems[b]).wait()

            @pl.when(g > 0)
            def _():
                pltpu.make_async_copy(
                    outs[b], o_hbm.at[pl.ds(0, BR)], osems[b]).wait()

            outs[b][...] = jnp.dot(ins[b][...], perm,
                                   preferred_element_type=jnp.float32)
            pltpu.make_async_copy(
                outs[b], o_hbm.at[pl.ds(i * BR, BR)], osems[b]).start()

            @pl.when(i + K < NBLK)
            def _():
                start_in(i + K, b)
        return ()

    lax.fori_loop(0, NBLK // K, group_body, ())
    for b in range(K):
        pltpu.make_async_copy(
            outs[b], o_hbm.at[pl.ds(0, BR)], osems[b]).wait()


def kernel(input, inv_indices):
    x2 = input.reshape(ROWS, IMW)
    perm = (inv_indices[None, :].astype(jnp.int32)
            == jnp.arange(IMW, dtype=jnp.int32)[:, None]).astype(jnp.float32)
    out = pl.pallas_call(
        _body,
        in_specs=[
            pl.BlockSpec(memory_space=pltpu.HBM),
            pl.BlockSpec(memory_space=pltpu.VMEM),
        ],
        out_specs=pl.BlockSpec(memory_space=pltpu.HBM),
        out_shape=jax.ShapeDtypeStruct((ROWS, IMW), input.dtype),
        scratch_shapes=(
            [pltpu.VMEM((BR, IMW), jnp.float32)] * 6
            + [pltpu.SemaphoreType.DMA] * 6
        ),
    )(x2, perm)
    return out.reshape(input.shape)
